# baseline (speedup 1.0000x reference)
import functools

import jax
import jax.numpy as jnp
from jax import lax
from jax.experimental import pallas as pl
from jax.experimental.pallas import tpu as pltpu
from jax.experimental.pallas import tpu_sc as plsc

D = 128
DW = D // 2
E = 640000
EH = E // 2
NW = 32
PER_W = EH // 16
C = 400
GR = 80
NCHUNK = PER_W // C
L = 16

_LOG1P = (
    1.0355222413e-08, 9.9999614843e-01, -4.9986781918e-01,
    3.3172977705e-01, -2.4038395607e-01, 1.6675330751e-01,
    -9.4099683279e-02, 3.5240213726e-02, -6.2208255655e-03,
)


def _sc_loss_partials(z, pos_x, neg_x):
  mesh = plsc.VectorSubcoreMesh(core_axis_name="c", subcore_axis_name="s")

  @functools.partial(
      pl.kernel,
      out_type=jax.ShapeDtypeStruct((NW, L), jnp.float32),
      mesh=mesh,
      compiler_params=pltpu.CompilerParams(
          needs_layout_passes=False, use_tc_tiling_on_sc=False),
      scratch_types=[
          pltpu.VMEM((2 * C,), jnp.int32),
          pltpu.VMEM((2 * C,), jnp.int32),
          pltpu.VMEM((C, DW), jnp.int32),
          pltpu.VMEM((C, DW), jnp.int32),
          pltpu.VMEM((C, DW), jnp.int32),
          pltpu.VMEM((C, DW), jnp.int32),
          pltpu.VMEM((C + L,), jnp.float32),
          pltpu.VMEM((L,), jnp.float32),
          pltpu.SemaphoreType.DMA,
          pltpu.SemaphoreType.DMA,
          pltpu.SemaphoreType.DMA,
          pltpu.SemaphoreType.DMA,
      ],
  )
  def k(z_hbm, pos_hbm, neg_hbm, out_hbm,
        ib0, ib1, srb0, drb0, srb1, drb1, svbuf, lbuf,
        xsem0, xsem1, gsem0, gsem1):
    wid = lax.axis_index("s") * 2 + lax.axis_index("c")
    is_pos = wid < 16
    ebase = jnp.where(is_pos, wid, wid - 16) * PER_W
    lblv = jnp.where(is_pos, 1.0, 0.0).astype(jnp.float32)
    lanes = lax.iota(jnp.int32, L)

    ib = (ib0, ib1)
    srb = (srb0, srb1)
    drb = (drb0, drb1)
    xsem = (xsem0, xsem1)
    gsem = (gsem0, gsem1)

    def _gather_parts(p):
      parts = []
      for t in range(C // GR):
        parts.append((ib[p].at[pl.ds(t * GR, GR)],
                      srb[p].at[pl.ds(t * GR, GR)]))
        parts.append((ib[p].at[pl.ds(C + t * GR, GR)],
                      drb[p].at[pl.ds(t * GR, GR)]))
      return parts

    def gather_start(p):
      for idx_row, dst in _gather_parts(p):
        pltpu.async_copy(z_hbm.at[idx_row], dst, gsem[p])

    def gather_wait(p):
      for idx_row, dst in _gather_parts(p):
        pltpu.make_async_copy(z_hbm.at[idx_row], dst, gsem[p]).wait()

    def idx_copy(p, i, start):
      base = ebase + i * C
      preds = ((pos_hbm, is_pos), (neg_hbm, jnp.logical_not(is_pos)))
      for x_hbm, pred in preds:
        @pl.when(pred)
        def _():
          for row in (0, 1):
            src = x_hbm.at[row, pl.ds(base, C)]
            dst = ib[p].at[pl.ds(row * C, C)]
            if start:
              pltpu.async_copy(src, dst, xsem[p])
            else:
              pltpu.sync_copy(src, dst)

    def idx_wait(p):
      for row in (0, 1):
        pltpu.make_async_copy(pos_hbm.at[0, pl.ds(0, C)],
                              ib[p].at[pl.ds(row * C, C)], xsem[p]).wait()

    idx_copy(0, 0, start=False)
    idx_copy(1, 1, start=True)
    gather_start(0)

    def half(i, p, lacc):
      q = 1 - p
      @pl.when(i + 1 < NCHUNK)
      def _():
        idx_wait(q)
        gather_start(q)
      gather_wait(p)
      @pl.when(i + 2 < NCHUNK)
      def _():
        idx_copy(p, i + 2, start=True)
      s, d = srb[p], drb[p]

      @plsc.parallel_loop(0, C, unroll=4)
      def _(r):
        prods = []
        for kk in range(DW // L):
          sp = plsc.bitcast(s[r, pl.ds(kk * L, L)], jnp.bfloat16)
          dp = plsc.bitcast(d[r, pl.ds(kk * L, L)], jnp.bfloat16)
          pa, pb = plsc.unpack(sp * dp, format=plsc.PackFormat.INTERLEAVED)
          prods += [pa, pb]
        while len(prods) > 1:
          prods = [prods[a] + prods[a + 1]
                   for a in range(0, len(prods), 2)]
        v = prods[0]
        for stp in (1, 2, 4, 8):
          v = v + jnp.take(v, lanes ^ stp)
        plsc.store_compressed(svbuf.at[pl.ds(r, L)], v, mask=lanes == 0)

      @plsc.parallel_loop(0, C // L, unroll=5, carry=lacc)
      def lacc(g, acc):
        scv = svbuf[pl.ds(g * L, L)]
        u = jnp.exp(-jnp.abs(scv))
        poly = jnp.float32(_LOG1P[-1])
        for cf in _LOG1P[-2::-1]:
          poly = poly * u + jnp.float32(cf)
        loss = jnp.maximum(scv, 0.0) - scv * lblv + poly
        return acc + loss

      return lacc

    @pl.loop(0, NCHUNK, step=2, init_carry=jnp.zeros((L,), jnp.float32))
    def total(j, lacc):
      lacc = half(j, 0, lacc)
      lacc = half(j + 1, 1, lacc)
      return lacc

    lbuf[...] = total
    pltpu.sync_copy(lbuf, out_hbm.at[wid])

  return k(z, pos_x, neg_x)


def _mean_tc(partials):
  def body(x_ref, o_ref):
    o_ref[...] = (jnp.sum(x_ref[...]) / E).reshape(1, 1)

  return pl.pallas_call(
      body,
      out_shape=jax.ShapeDtypeStruct((1, 1), jnp.float32),
  )(partials)


@jax.jit
def kernel(z, pos_edge_index, neg_edge_index):
  pos_x = jnp.asarray(pos_edge_index, jnp.int32)
  neg_x = jnp.asarray(neg_edge_index, jnp.int32)
  u = jax.lax.bitcast_convert_type(z, jnp.uint32)
  r = (u + jnp.uint32(0x7FFF) + ((u >> 16) & jnp.uint32(1))) >> 16
  z32 = jax.lax.bitcast_convert_type(
      r[:, :DW] | (r[:, DW:] << 16), jnp.int32)
  partials = _sc_loss_partials(z32, pos_x, neg_x)
  loss = _mean_tc(partials.reshape(NW * L // 128, 128))
  return loss[0, 0]


# speedup vs baseline: 1.1471x; 1.1471x over previous
import functools

import jax
import jax.numpy as jnp
from jax import lax
from jax.experimental import pallas as pl
from jax.experimental.pallas import tpu as pltpu
from jax.experimental.pallas import tpu_sc as plsc

D = 128
DW = D // 2
E = 640000
EH = E // 2
NW = 32
PER_W = EH // 16
C = 400
GR = 80
NCHUNK = PER_W // C
L = 16

_LOG1P = (
    1.0355222413e-08, 9.9999614843e-01, -4.9986781918e-01,
    3.3172977705e-01, -2.4038395607e-01, 1.6675330751e-01,
    -9.4099683279e-02, 3.5240213726e-02, -6.2208255655e-03,
)


def _sc_loss_partials(z, pos_x, neg_x):
  mesh = plsc.VectorSubcoreMesh(core_axis_name="c", subcore_axis_name="s")

  @functools.partial(
      pl.kernel,
      out_type=jax.ShapeDtypeStruct((NW, L), jnp.float32),
      mesh=mesh,
      compiler_params=pltpu.CompilerParams(
          needs_layout_passes=False, use_tc_tiling_on_sc=False),
      scratch_types=[
          pltpu.VMEM((2 * C,), jnp.int32),
          pltpu.VMEM((2 * C,), jnp.int32),
          pltpu.VMEM((C, DW), jnp.int32),
          pltpu.VMEM((C, DW), jnp.int32),
          pltpu.VMEM((C, DW), jnp.int32),
          pltpu.VMEM((C, DW), jnp.int32),
          pltpu.VMEM((C * L,), jnp.float32),
          pltpu.VMEM((L,), jnp.float32),
          pltpu.SemaphoreType.DMA,
          pltpu.SemaphoreType.DMA,
          pltpu.SemaphoreType.DMA,
          pltpu.SemaphoreType.DMA,
      ],
  )
  def k(z_hbm, pos_hbm, neg_hbm, out_hbm,
        ib0, ib1, srb0, drb0, srb1, drb1, accbuf, lbuf,
        xsem0, xsem1, gsem0, gsem1):
    wid = lax.axis_index("s") * 2 + lax.axis_index("c")
    is_pos = wid < 16
    ebase = jnp.where(is_pos, wid, wid - 16) * PER_W
    lblv = jnp.where(is_pos, 1.0, 0.0).astype(jnp.float32)
    lanes = lax.iota(jnp.int32, L)

    ib = (ib0, ib1)
    srb = (srb0, srb1)
    drb = (drb0, drb1)
    xsem = (xsem0, xsem1)
    gsem = (gsem0, gsem1)

    def _gather_parts(p):
      parts = []
      for t in range(C // GR):
        parts.append((ib[p].at[pl.ds(t * GR, GR)],
                      srb[p].at[pl.ds(t * GR, GR)]))
        parts.append((ib[p].at[pl.ds(C + t * GR, GR)],
                      drb[p].at[pl.ds(t * GR, GR)]))
      return parts

    def gather_start(p):
      for idx_row, dst in _gather_parts(p):
        pltpu.async_copy(z_hbm.at[idx_row], dst, gsem[p])

    def gather_wait(p):
      for idx_row, dst in _gather_parts(p):
        pltpu.make_async_copy(z_hbm.at[idx_row], dst, gsem[p]).wait()

    def idx_copy(p, i, start):
      base = ebase + i * C
      preds = ((pos_hbm, is_pos), (neg_hbm, jnp.logical_not(is_pos)))
      for x_hbm, pred in preds:
        @pl.when(pred)
        def _():
          for row in (0, 1):
            src = x_hbm.at[row, pl.ds(base, C)]
            dst = ib[p].at[pl.ds(row * C, C)]
            if start:
              pltpu.async_copy(src, dst, xsem[p])
            else:
              pltpu.sync_copy(src, dst)

    def idx_wait(p):
      for row in (0, 1):
        pltpu.make_async_copy(pos_hbm.at[0, pl.ds(0, C)],
                              ib[p].at[pl.ds(row * C, C)], xsem[p]).wait()

    idx_copy(0, 0, start=False)
    idx_copy(1, 1, start=True)
    gather_start(0)

    def half(i, p, lacc):
      q = 1 - p
      @pl.when(i + 1 < NCHUNK)
      def _():
        idx_wait(q)
        gather_start(q)
      gather_wait(p)
      @pl.when(i + 2 < NCHUNK)
      def _():
        idx_copy(p, i + 2, start=True)
      s, d = srb[p], drb[p]

      @plsc.parallel_loop(0, C, unroll=4)
      def _(r):
        prods = []
        for kk in range(DW // L):
          sp = plsc.bitcast(s[r, pl.ds(kk * L, L)], jnp.bfloat16)
          dp = plsc.bitcast(d[r, pl.ds(kk * L, L)], jnp.bfloat16)
          pa, pb = plsc.unpack(sp * dp, format=plsc.PackFormat.INTERLEAVED)
          prods += [pa, pb]
        while len(prods) > 1:
          prods = [prods[a] + prods[a + 1]
                   for a in range(0, len(prods), 2)]
        accbuf[pl.ds(r * L, L)] = prods[0]

      @plsc.parallel_loop(0, C // L, unroll=5, carry=lacc)
      def lacc(g, acc):
        scv = None
        for t in range(L):
          dg = lanes * L + ((lanes + t) & (L - 1)) + g * (L * L)
          v = plsc.load_gather(accbuf, [dg])
          scv = v if scv is None else scv + v
        u = jnp.exp(-jnp.abs(scv))
        poly = jnp.float32(_LOG1P[-1])
        for cf in _LOG1P[-2::-1]:
          poly = poly * u + jnp.float32(cf)
        loss = jnp.maximum(scv, 0.0) - scv * lblv + poly
        return acc + loss

      return lacc

    @pl.loop(0, NCHUNK, step=2, init_carry=jnp.zeros((L,), jnp.float32))
    def total(j, lacc):
      lacc = half(j, 0, lacc)
      lacc = half(j + 1, 1, lacc)
      return lacc

    lbuf[...] = total
    pltpu.sync_copy(lbuf, out_hbm.at[wid])

  return k(z, pos_x, neg_x)


def _mean_tc(partials):
  def body(x_ref, o_ref):
    o_ref[...] = (jnp.sum(x_ref[...]) / E).reshape(1, 1)

  return pl.pallas_call(
      body,
      out_shape=jax.ShapeDtypeStruct((1, 1), jnp.float32),
  )(partials)


@jax.jit
def kernel(z, pos_edge_index, neg_edge_index):
  pos_x = jnp.asarray(pos_edge_index, jnp.int32)
  neg_x = jnp.asarray(neg_edge_index, jnp.int32)
  u = jax.lax.bitcast_convert_type(z, jnp.uint32)
  r = (u + jnp.uint32(0x7FFF) + ((u >> 16) & jnp.uint32(1))) >> 16
  z32 = jax.lax.bitcast_convert_type(
      r[:, :DW] | (r[:, DW:] << 16), jnp.int32)
  partials = _sc_loss_partials(z32, pos_x, neg_x)
  loss = _mean_tc(partials.reshape(NW * L // 128, 128))
  return loss[0, 0]
